# baseline (speedup 1.0000x reference)
import functools

import jax
import jax.numpy as jnp
from jax import lax
from jax.experimental import pallas as pl
from jax.experimental.pallas import tpu as pltpu
from jax.experimental.pallas import tpu_sc as plsc

N = 10000
D = 128
E = 320000

NC = 2
NS = 16
NW = NC * NS

CHUNK = 128
NCHUNK = 80
G = 8
NG = NCHUNK // G
EPW = CHUNK * NCHUNK
EP = EPW * NW
NP = 10240
SHARD = NP // NS

_mesh = plsc.VectorSubcoreMesh(core_axis_name="c", subcore_axis_name="s")


def _msg_body(xs_hbm, srcr, dstr, zeros_hbm, out_hbm,
              src_v, dst0, dst1, rows0, rows1, shared_out,
              sem0, sem1, semd0, semd1):
    c = lax.axis_index("c")
    s = lax.axis_index("s")
    wid = s * NC + c
    pltpu.sync_copy(zeros_hbm, shared_out.at[pl.ds(s * SHARD, SHARD)])
    pltpu.sync_copy(srcr.at[wid], src_v)
    plsc.subcore_barrier()

    rows = (rows0, rows1)
    sems = (sem0, sem1)

    pltpu.async_copy(dstr.at[wid, 0], dst0, semd0)
    pltpu.async_copy(dstr.at[wid, 1], dst1, semd1)
    pltpu.async_copy(xs_hbm.at[src_v.at[0]], rows0, sem0)

    def step(j, b, dst_g, k):
        nb = 1 - b
        pltpu.make_async_copy(xs_hbm.at[src_v.at[j]], rows[b], sems[b]).wait()

        @pl.when(j + 1 < NCHUNK)
        def _():
            pltpu.async_copy(xs_hbm.at[src_v.at[j + 1]], rows[nb], sems[nb])

        pltpu.sync_copy(rows[b], shared_out.at[dst_g.at[k]], add=True)

    def group_block(g, dst_g, semd):
        pltpu.make_async_copy(dstr.at[wid, g], dst_g, semd).wait()

        def inner(jj, carry):
            j = g * G + 2 * jj
            step(j, 0, dst_g, 2 * jj)
            step(j + 1, 1, dst_g, 2 * jj + 1)
            return carry

        lax.fori_loop(0, G // 2, inner, 0, unroll=False)

        @pl.when(g + 2 < NG)
        def _():
            pltpu.async_copy(dstr.at[wid, g + 2], dst_g, semd)

    def body(gg, carry):
        group_block(gg * 2, dst0, semd0)
        group_block(gg * 2 + 1, dst1, semd1)
        return carry

    lax.fori_loop(0, NG // 2, body, 0, unroll=False)
    plsc.subcore_barrier()
    pltpu.sync_copy(shared_out.at[pl.ds(s * SHARD, SHARD)],
                    out_hbm.at[c, pl.ds(s * SHARD, SHARD)])


def _deg_body(ones_hbm, srcr, zeros_hbm, out_hbm,
              src_v, rows_v, shared_out):
    c = lax.axis_index("c")
    s = lax.axis_index("s")
    wid = s * NC + c
    pltpu.sync_copy(zeros_hbm, shared_out.at[pl.ds(s * SHARD, SHARD)])
    pltpu.sync_copy(srcr.at[wid], src_v)
    pltpu.sync_copy(ones_hbm, rows_v)
    plsc.subcore_barrier()

    def body(j, carry):
        pltpu.sync_copy(rows_v, shared_out.at[src_v.at[j]], add=True)
        return carry

    lax.fori_loop(0, NCHUNK, body, 0, unroll=False)
    plsc.subcore_barrier()
    pltpu.sync_copy(shared_out.at[pl.ds(s * SHARD, SHARD)],
                    out_hbm.at[c, pl.ds(s * SHARD, SHARD)])


_msg_pass = pl.kernel(
    _msg_body,
    out_type=jax.ShapeDtypeStruct((NC, NP, D), jnp.float32),
    mesh=_mesh,
    scratch_types=[
        pltpu.VMEM((NCHUNK, CHUNK), jnp.int32),
        pltpu.VMEM((G, CHUNK), jnp.int32),
        pltpu.VMEM((G, CHUNK), jnp.int32),
        pltpu.VMEM((CHUNK, D), jnp.float32),
        pltpu.VMEM((CHUNK, D), jnp.float32),
        pltpu.VMEM_SHARED((NP, D), jnp.float32),
        pltpu.SemaphoreType.DMA,
        pltpu.SemaphoreType.DMA,
        pltpu.SemaphoreType.DMA,
        pltpu.SemaphoreType.DMA,
    ],
)

_deg_pass = pl.kernel(
    _deg_body,
    out_type=jax.ShapeDtypeStruct((NC, NP, D), jnp.float32),
    mesh=_mesh,
    scratch_types=[
        pltpu.VMEM((NCHUNK, CHUNK), jnp.int32),
        pltpu.VMEM((CHUNK, D), jnp.float32),
        pltpu.VMEM_SHARED((NP, D), jnp.float32),
    ],
)


def _dv(degp):
    return lax.rsqrt(1.0 + degp[0][:, 0:1] + degp[1][:, 0:1])


def _tc_first_body(x_ref, w_ref, deg_ref, xs_ref):
    dv = _dv(deg_ref)
    xs = dv * jnp.dot(x_ref[...], w_ref[...],
                      preferred_element_type=jnp.float32)
    xs_ref[0] = xs
    xs_ref[1] = xs


def _tc_mid_body(p_ref, xs_ref, deg_ref, b_ref, w_ref, o_ref):
    dv = _dv(deg_ref)
    h = dv * (p_ref[0] + p_ref[1] + xs_ref[0]) + b_ref[...]
    h = jnp.maximum(h, 0.0)
    xs = dv * jnp.dot(h, w_ref[...], preferred_element_type=jnp.float32)
    o_ref[0] = xs
    o_ref[1] = xs


def _tc_last_body(p_ref, xs_ref, deg_ref, b_ref, o_ref):
    dv = _dv(deg_ref)
    o_ref[...] = dv * (p_ref[0] + p_ref[1] + xs_ref[0]) + b_ref[...]


BLK = 512
GRID = NP // BLK

_row_spec = pl.BlockSpec((BLK, D), lambda i: (i, 0))
_p_spec = pl.BlockSpec((NC, BLK, D), lambda i: (0, i, 0))
_deg_spec = pl.BlockSpec((NC, BLK, D), lambda i: (0, i, 0))
_xsr_spec = pl.BlockSpec((1, BLK, D), lambda i: (0, i, 0))
_xsw_spec = pl.BlockSpec((NC, BLK, D), lambda i: (0, i, 0))
_w_spec = pl.BlockSpec((D, D), lambda i: (0, 0))
_b_spec = pl.BlockSpec((1, D), lambda i: (0, 0))
_out_struct = jax.ShapeDtypeStruct((NP, D), jnp.float32)
_xs_struct = jax.ShapeDtypeStruct((NC, NP, D), jnp.float32)

_tc_first = pl.pallas_call(
    _tc_first_body,
    grid=(GRID,),
    in_specs=[_row_spec, _w_spec, _deg_spec],
    out_specs=_xsw_spec,
    out_shape=_xs_struct,
)

_tc_mid = pl.pallas_call(
    _tc_mid_body,
    grid=(GRID,),
    in_specs=[_p_spec, _xsr_spec, _deg_spec, _b_spec, _w_spec],
    out_specs=_xsw_spec,
    out_shape=_xs_struct,
)

_tc_last = pl.pallas_call(
    _tc_last_body,
    grid=(GRID,),
    in_specs=[_p_spec, _xsr_spec, _deg_spec, _b_spec],
    out_specs=_row_spec,
    out_shape=_out_struct,
)


@jax.jit
def _run(x, edge_index, W1, b1, W2, b2, W3, b3):
    pad = EP - E
    src = jnp.concatenate([edge_index[0], jnp.full((pad,), N, jnp.int32)])
    dst = jnp.concatenate([edge_index[1], jnp.full((pad,), N, jnp.int32)])
    srcr0 = src.reshape(NW, NCHUNK, CHUNK)
    srcr = srcr0 + (NP * (jnp.arange(NW, dtype=jnp.int32) % NC))[:, None, None]
    dstr = dst.reshape(NW, NG, G, CHUNK)

    xp = jnp.zeros((NP, D), jnp.float32).at[:N].set(x)
    zeros128 = jnp.zeros((SHARD, D), jnp.float32)
    ones128 = jnp.ones((CHUNK, D), jnp.float32)

    degp = _deg_pass(ones128, srcr0, zeros128)

    xs = _tc_first(xp, W1, degp)
    p = _msg_pass(xs.reshape(NC * NP, D), srcr, dstr, zeros128)
    xs = _tc_mid(p, xs, degp, b1.reshape(1, D), W2)
    p = _msg_pass(xs.reshape(NC * NP, D), srcr, dstr, zeros128)
    xs = _tc_mid(p, xs, degp, b2.reshape(1, D), W3)
    p = _msg_pass(xs.reshape(NC * NP, D), srcr, dstr, zeros128)
    out = _tc_last(p, xs, degp, b3.reshape(1, D))
    return out[:N]


def kernel(x, edge_index, cache_name, W1, b1, W2, b2, W3, b3):
    return _run(x, edge_index, W1, b1, W2, b2, W3, b3)


# speedup vs baseline: 1.0653x; 1.0653x over previous
import jax
import jax.numpy as jnp
from jax import lax
from jax.experimental import pallas as pl
from jax.experimental.pallas import tpu as pltpu
from jax.experimental.pallas import tpu_sc as plsc

N = 10000
D = 128
E = 320000

NC = 2
NS = 16
NW = NC * NS

CHUNK = 128
TCH = 160
KF = 112
KS = TCH - KF
G = 8
EP = NS * TCH * CHUNK
NCHUNK_DEG = EP // (NW * CHUNK)
NP = 10112
SHARD = NP // NS

FAST_CORE = 0

_mesh = plsc.VectorSubcoreMesh(core_axis_name="c", subcore_axis_name="s")


def _msg_body(xs_hbm, srcr, dstr, zeros_hbm, out_hbm,
              src_v, dst0, dst1, rows0, rows1, shared_out,
              sem0, sem1, semd0, semd1):
    c = lax.axis_index("c")
    s = lax.axis_index("s")
    pltpu.sync_copy(zeros_hbm, shared_out.at[pl.ds(s * SHARD, SHARD)])
    plsc.subcore_barrier()

    rows = (rows0, rows1)
    sems = (sem0, sem1)

    def run(start, nch, pipelined):
        ng = nch // G
        g0 = start // G
        pltpu.sync_copy(srcr.at[s, pl.ds(start, nch)],
                        src_v.at[pl.ds(0, nch)])
        pltpu.async_copy(dstr.at[s, g0], dst0, semd0)
        pltpu.async_copy(dstr.at[s, g0 + 1], dst1, semd1)
        if pipelined:
            pltpu.async_copy(xs_hbm.at[src_v.at[0]], rows0, sem0)

        def step_pipe(j, b, dst_g, k):
            nb = 1 - b
            pltpu.make_async_copy(xs_hbm.at[src_v.at[j]], rows[b],
                                  sems[b]).wait()

            @pl.when(j + 1 < nch)
            def _():
                pltpu.async_copy(xs_hbm.at[src_v.at[j + 1]], rows[nb],
                                 sems[nb])

            pltpu.sync_copy(rows[b], shared_out.at[dst_g.at[k]], add=True)

        def step_ser(j, dst_g, k):
            pltpu.async_copy(xs_hbm.at[src_v.at[j]], rows0, sem0).wait()
            pltpu.sync_copy(rows0, shared_out.at[dst_g.at[k]], add=True)

        def gblock(g, dst_g, semd):
            pltpu.make_async_copy(dstr.at[s, g0 + g], dst_g, semd).wait()

            def inner(jj, carry):
                j = g * G + 2 * jj
                if pipelined:
                    step_pipe(j, 0, dst_g, 2 * jj)
                    step_pipe(j + 1, 1, dst_g, 2 * jj + 1)
                else:
                    step_ser(j, dst_g, 2 * jj)
                    step_ser(j + 1, dst_g, 2 * jj + 1)
                return carry

            lax.fori_loop(0, G // 2, inner, 0, unroll=False)

            @pl.when(g + 2 < ng)
            def _():
                pltpu.async_copy(dstr.at[s, g0 + g + 2], dst_g, semd)

        def pair(gg, carry):
            gblock(gg * 2, dst0, semd0)
            gblock(gg * 2 + 1, dst1, semd1)
            return carry

        lax.fori_loop(0, ng // 2, pair, 0, unroll=False)

    @pl.when(c == FAST_CORE)
    def _():
        run(0, KF, True)

    @pl.when(c == 1 - FAST_CORE)
    def _():
        run(KF, KS, False)

    plsc.subcore_barrier()
    pltpu.sync_copy(shared_out.at[pl.ds(s * SHARD, SHARD)],
                    out_hbm.at[c, pl.ds(s * SHARD, SHARD)])


def _deg_body(ones_hbm, srcr, zeros_hbm, out_hbm,
              src_v, rows_v, shared_out):
    c = lax.axis_index("c")
    s = lax.axis_index("s")
    wid = s * NC + c
    pltpu.sync_copy(zeros_hbm, shared_out.at[pl.ds(s * SHARD, SHARD)])
    pltpu.sync_copy(srcr.at[wid], src_v)
    pltpu.sync_copy(ones_hbm, rows_v)
    plsc.subcore_barrier()

    def body(j, carry):
        pltpu.sync_copy(rows_v, shared_out.at[src_v.at[j]], add=True)
        return carry

    lax.fori_loop(0, NCHUNK_DEG, body, 0, unroll=False)
    plsc.subcore_barrier()
    pltpu.sync_copy(shared_out.at[pl.ds(s * SHARD, SHARD)],
                    out_hbm.at[c, pl.ds(s * SHARD, SHARD)])


_msg_pass = pl.kernel(
    _msg_body,
    out_type=jax.ShapeDtypeStruct((NC, NP, D), jnp.float32),
    mesh=_mesh,
    scratch_types=[
        pltpu.VMEM((KF, CHUNK), jnp.int32),
        pltpu.VMEM((G, CHUNK), jnp.int32),
        pltpu.VMEM((G, CHUNK), jnp.int32),
        pltpu.VMEM((CHUNK, D), jnp.float32),
        pltpu.VMEM((CHUNK, D), jnp.float32),
        pltpu.VMEM_SHARED((NP, D), jnp.float32),
        pltpu.SemaphoreType.DMA,
        pltpu.SemaphoreType.DMA,
        pltpu.SemaphoreType.DMA,
        pltpu.SemaphoreType.DMA,
    ],
)

_deg_pass = pl.kernel(
    _deg_body,
    out_type=jax.ShapeDtypeStruct((NC, NP, D), jnp.float32),
    mesh=_mesh,
    scratch_types=[
        pltpu.VMEM((NCHUNK_DEG, CHUNK), jnp.int32),
        pltpu.VMEM((CHUNK, D), jnp.float32),
        pltpu.VMEM_SHARED((NP, D), jnp.float32),
    ],
)


def _dv(degp):
    return lax.rsqrt(1.0 + degp[0][:, 0:1] + degp[1][:, 0:1])


def _tc_first_body(x_ref, w_ref, deg_ref, xs_ref):
    dv = _dv(deg_ref)
    xs_ref[...] = dv * jnp.dot(x_ref[...], w_ref[...],
                               preferred_element_type=jnp.float32)


def _tc_mid_body(p_ref, xs_ref, deg_ref, b_ref, w_ref, o_ref):
    dv = _dv(deg_ref)
    h = dv * (p_ref[0] + p_ref[1] + xs_ref[...]) + b_ref[...]
    h = jnp.maximum(h, 0.0)
    o_ref[...] = dv * jnp.dot(h, w_ref[...],
                              preferred_element_type=jnp.float32)


def _tc_last_body(p_ref, xs_ref, deg_ref, b_ref, o_ref):
    dv = _dv(deg_ref)
    o_ref[...] = dv * (p_ref[0] + p_ref[1] + xs_ref[...]) + b_ref[...]


BLK = 632
GRID = NP // BLK

_row_spec = pl.BlockSpec((BLK, D), lambda i: (i, 0))
_p_spec = pl.BlockSpec((NC, BLK, D), lambda i: (0, i, 0))
_deg_spec = pl.BlockSpec((NC, BLK, D), lambda i: (0, i, 0))
_w_spec = pl.BlockSpec((D, D), lambda i: (0, 0))
_b_spec = pl.BlockSpec((1, D), lambda i: (0, 0))
_out_struct = jax.ShapeDtypeStruct((NP, D), jnp.float32)

_tc_first = pl.pallas_call(
    _tc_first_body,
    grid=(GRID,),
    in_specs=[_row_spec, _w_spec, _deg_spec],
    out_specs=_row_spec,
    out_shape=_out_struct,
)

_tc_mid = pl.pallas_call(
    _tc_mid_body,
    grid=(GRID,),
    in_specs=[_p_spec, _row_spec, _deg_spec, _b_spec, _w_spec],
    out_specs=_row_spec,
    out_shape=_out_struct,
)

_tc_last = pl.pallas_call(
    _tc_last_body,
    grid=(GRID,),
    in_specs=[_p_spec, _row_spec, _deg_spec, _b_spec],
    out_specs=_row_spec,
    out_shape=_out_struct,
)


@jax.jit
def _run(x, edge_index, W1, b1, W2, b2, W3, b3):
    pad = EP - E
    src = jnp.concatenate([edge_index[0], jnp.full((pad,), N, jnp.int32)])
    dst = jnp.concatenate([edge_index[1], jnp.full((pad,), N, jnp.int32)])
    srcr_deg = src.reshape(NW, NCHUNK_DEG, CHUNK)
    srcr = src.reshape(NS, TCH, CHUNK)
    dstr = dst.reshape(NS, TCH // G, G, CHUNK)

    xp = jnp.zeros((NP, D), jnp.float32).at[:N].set(x)
    zeros128 = jnp.zeros((SHARD, D), jnp.float32)
    ones128 = jnp.ones((CHUNK, D), jnp.float32)

    degp = _deg_pass(ones128, srcr_deg, zeros128)

    xs = _tc_first(xp, W1, degp)
    p = _msg_pass(xs, srcr, dstr, zeros128)
    xs = _tc_mid(p, xs, degp, b1.reshape(1, D), W2)
    p = _msg_pass(xs, srcr, dstr, zeros128)
    xs = _tc_mid(p, xs, degp, b2.reshape(1, D), W3)
    p = _msg_pass(xs, srcr, dstr, zeros128)
    out = _tc_last(p, xs, degp, b3.reshape(1, D))
    return out[:N]


def kernel(x, edge_index, cache_name, W1, b1, W2, b2, W3, b3):
    return _run(x, edge_index, W1, b1, W2, b2, W3, b3)
